# baseline (speedup 1.0000x reference)
import functools

import jax
import jax.numpy as jnp
from jax import lax
from jax.experimental import pallas as pl
from jax.experimental.pallas import tpu as pltpu
from jax.experimental.pallas import tpu_sc as plsc

N_NODES = 10000
N_EDGES = 320000
N_GRAPHS = 16

NUM_CORES = 2
NUM_TILES = 16
LANE = 128
DC = 128
G = 16

ROWS_PER_TILE = 632
N_ACC = ROWS_PER_TILE * NUM_TILES
N_PADROWS = 16

CHUNKS_C = 160
E_PAD_C = NUM_TILES * CHUNKS_C * LANE

CHUNKS_E = 80
E_PAD_E = NUM_CORES * NUM_TILES * CHUNKS_E * LANE

_MESH = plsc.VectorSubcoreMesh(core_axis_name="c", subcore_axis_name="s")


def _make_deg_kernel():
    B = 8
    assert CHUNKS_E % B == 0

    @functools.partial(
        pl.kernel,
        out_type=jax.ShapeDtypeStruct((NUM_CORES * N_ACC,), jnp.float32),
        mesh=_MESH,
        scratch_types=[
            pltpu.VMEM((CHUNKS_E, LANE), jnp.int32),
            pltpu.VMEM((LANE,), jnp.float32),
            pltpu.VMEM((ROWS_PER_TILE,), jnp.float32),
            pltpu.VMEM_SHARED((N_ACC,), jnp.float32),
            pltpu.SemaphoreType.DMA,
        ],
    )
    def deg_kernel(dst_hbm, ones_hbm, zeros_hbm, out_hbm, dst_v, ones_v,
                   tmp_v, acc_sh, sem):
        c = lax.axis_index("c")
        s = lax.axis_index("s")
        row0 = s * ROWS_PER_TILE
        my_rows = pl.ds(row0, ROWS_PER_TILE)
        pltpu.sync_copy(zeros_hbm.at[my_rows], tmp_v)
        pltpu.sync_copy(tmp_v, acc_sh.at[my_rows])
        pltpu.sync_copy(dst_hbm.at[c, s], dst_v)
        pltpu.sync_copy(ones_hbm, ones_v)
        plsc.subcore_barrier()

        @pl.loop(0, CHUNKS_E, step=B)
        def _batch(k0):
            for j in range(B):
                pltpu.async_copy(ones_v, acc_sh.at[dst_v.at[k0 + j]], sem,
                                 add=True)
            for j in range(B):
                pltpu.make_async_copy(
                    ones_v, acc_sh.at[dst_v.at[k0 + j]], sem).wait()

        plsc.subcore_barrier()
        pltpu.sync_copy(acc_sh.at[my_rows], tmp_v)
        pltpu.sync_copy(tmp_v,
                        out_hbm.at[pl.ds(c * N_ACC + row0, ROWS_PER_TILE)])

    return deg_kernel


def _make_prop_kernel(nchunks):
    ngroups = nchunks // G
    assert ngroups * G == nchunks

    @functools.partial(
        pl.kernel,
        out_type=jax.ShapeDtypeStruct((NUM_CORES, N_ACC, DC), jnp.float32),
        mesh=_MESH,
        scratch_types=[
            pltpu.VMEM((2, G, LANE), jnp.int32),
            pltpu.VMEM((2, G, LANE), jnp.int32),
            pltpu.VMEM((2, LANE, DC), jnp.float32),
            pltpu.VMEM_SHARED((N_ACC, DC), jnp.float32),
            pltpu.SemaphoreType.DMA,
            pltpu.SemaphoreType.DMA,
            pltpu.SemaphoreType.DMA,
            pltpu.SemaphoreType.DMA,
            pltpu.SemaphoreType.DMA,
        ],
    )
    def prop_kernel(u_hbm, src_hbm, dst_hbm, zeros_hbm, out_hbm,
                    src_v, dst_v, rows_v, acc_sh, gsem0, gsem1, ssem0, ssem1,
                    isem):
        c = lax.axis_index("c")
        s = lax.axis_index("s")
        row0 = s * ROWS_PER_TILE
        my_rows = pl.ds(row0, ROWS_PER_TILE)
        pltpu.sync_copy(zeros_hbm.at[my_rows], acc_sh.at[my_rows])
        pltpu.sync_copy(src_hbm.at[c, s, pl.ds(0, G)], src_v.at[0])
        pltpu.sync_copy(dst_hbm.at[c, s, pl.ds(0, G)], dst_v.at[0])
        plsc.subcore_barrier()

        gsems = (gsem0, gsem1)
        ssems = (ssem0, ssem1)
        pltpu.async_copy(u_hbm.at[src_v.at[0, 0]], rows_v.at[0], gsem0)

        @pl.loop(0, ngroups)
        def _group(g):
            gi = lax.rem(g, 2)
            for j in range(G):
                k = g * G + j
                b = j % 2

                @pl.when(k > 0)
                def _wait_prev_scatter():
                    pltpu.make_async_copy(
                        rows_v.at[1 - b], acc_sh.at[dst_v.at[gi, j]],
                        ssems[1 - b]).wait()

                if j == 0:
                    @pl.when(g + 1 < ngroups)
                    def _load_next():
                        off = (g + 1) * G
                        pltpu.async_copy(
                            src_hbm.at[c, s, pl.ds(off, G)],
                            src_v.at[1 - gi], isem)
                        pltpu.async_copy(
                            dst_hbm.at[c, s, pl.ds(off, G)],
                            dst_v.at[1 - gi], isem)

                if j == G - 1:
                    @pl.when(g + 1 < ngroups)
                    def _wait_idx():
                        pltpu.make_async_copy(
                            src_hbm.at[c, s, pl.ds(0, G)], src_v.at[0],
                            isem).wait()
                        pltpu.make_async_copy(
                            dst_hbm.at[c, s, pl.ds(0, G)], dst_v.at[0],
                            isem).wait()

                @pl.when(k + 1 < nchunks)
                def _next_gather():
                    if j == G - 1:
                        idx_row = src_v.at[1 - gi, 0]
                    else:
                        idx_row = src_v.at[gi, j + 1]
                    pltpu.async_copy(u_hbm.at[idx_row], rows_v.at[1 - b],
                                     gsems[1 - b], priority=1)

                pltpu.make_async_copy(
                    u_hbm.at[src_v.at[gi, j]], rows_v.at[b], gsems[b]).wait()
                pltpu.async_copy(rows_v.at[b], acc_sh.at[dst_v.at[gi, j]],
                                 ssems[b], add=True)

        bl = (nchunks - 1) % 2
        pltpu.make_async_copy(
            rows_v.at[bl], acc_sh.at[dst_v.at[0, 0]], ssems[bl]).wait()
        plsc.subcore_barrier()
        pltpu.sync_copy(acc_sh.at[my_rows], out_hbm.at[c, my_rows])

    return prop_kernel


_deg_kernel = _make_deg_kernel()
_prop_edge = _make_prop_kernel(CHUNKS_E)
_prop_col = _make_prop_kernel(CHUNKS_C)


_BT = 1000
_NB = N_NODES // _BT


def _dinv_of(degp_ref):
    deg = degp_ref[0, 0, :] + 1.0
    return lax.rsqrt(jnp.maximum(deg, 1.0))


def _layernorm(h, g_ref, be_ref):
    mu = jnp.mean(h, axis=1, keepdims=True)
    var = jnp.mean((h - mu) ** 2, axis=1, keepdims=True)
    return (h - mu) * lax.rsqrt(var + 1e-5) * g_ref[0, :] + be_ref[0, :]


def _scale_body(degp_ref, x_ref, out_ref):
    dinv = _dinv_of(degp_ref)
    out_ref[...] = x_ref[...] * dinv[:, None]


def _tc_scale(degp, x):
    return pl.pallas_call(
        _scale_body,
        grid=(_NB,),
        in_specs=[
            pl.BlockSpec((1, 1, _BT), lambda i: (i, 0, 0)),
            pl.BlockSpec((_BT, DC), lambda i: (i, 0)),
        ],
        out_specs=pl.BlockSpec((_BT, DC), lambda i: (i, 0)),
        out_shape=jax.ShapeDtypeStruct((N_NODES, DC), jnp.float32),
    )(degp, x)


def _layer1_body(degp_ref, s_ref, u_ref, w_ref, b_ref, g_ref, be_ref, out_ref):
    dinv = _dinv_of(degp_ref)
    a = (s_ref[0] + s_ref[1] + u_ref[...]) * dinv[:, None]
    h = jnp.dot(a, w_ref[...], preferred_element_type=jnp.float32) + b_ref[0, :]
    h = jnp.maximum(h, 0.0)
    un = _layernorm(h, g_ref, be_ref) * dinv[:, None]
    out_ref[0] = un[:, :DC]
    out_ref[1] = un[:, DC:]


def _tc_layer1(degp, s, u, w, b, g, be):
    dhid = w.shape[1]
    return pl.pallas_call(
        _layer1_body,
        grid=(_NB,),
        in_specs=[
            pl.BlockSpec((1, 1, _BT), lambda i: (i, 0, 0)),
            pl.BlockSpec((2, _BT, DC), lambda i: (0, i, 0)),
            pl.BlockSpec((_BT, DC), lambda i: (i, 0)),
            pl.BlockSpec((DC, dhid), lambda i: (0, 0)),
            pl.BlockSpec((1, dhid), lambda i: (0, 0)),
            pl.BlockSpec((1, dhid), lambda i: (0, 0)),
            pl.BlockSpec((1, dhid), lambda i: (0, 0)),
        ],
        out_specs=pl.BlockSpec((2, _BT, DC), lambda i: (0, i, 0)),
        out_shape=jax.ShapeDtypeStruct((2, N_NODES, DC), jnp.float32),
    )(degp, s, u, w, b, g, be)


def _layer3_body(degp_ref, s_ref, u_ref, w2_ref, b2_ref, g2_ref, be2_ref,
                 w3_ref, out_ref):
    dinv = _dinv_of(degp_ref)
    a = jnp.concatenate([s_ref[0] + u_ref[0], s_ref[1] + u_ref[1]], axis=1)
    a = a * dinv[:, None]
    h = jnp.dot(a, w2_ref[...], preferred_element_type=jnp.float32) + b2_ref[0, :]
    h = jnp.maximum(h, 0.0)
    h = _layernorm(h, g2_ref, be2_ref)
    t = jnp.dot(h, w3_ref[...], preferred_element_type=jnp.float32)
    out_ref[...] = t * dinv[:, None]


def _tc_layer3(degp, s, u, w2, b2, g2, be2, w3):
    dhid = w2.shape[1]
    return pl.pallas_call(
        _layer3_body,
        grid=(_NB,),
        in_specs=[
            pl.BlockSpec((1, 1, _BT), lambda i: (i, 0, 0)),
            pl.BlockSpec((2, _BT, DC), lambda i: (0, i, 0)),
            pl.BlockSpec((2, _BT, DC), lambda i: (0, i, 0)),
            pl.BlockSpec((2 * DC, dhid), lambda i: (0, 0)),
            pl.BlockSpec((1, dhid), lambda i: (0, 0)),
            pl.BlockSpec((1, dhid), lambda i: (0, 0)),
            pl.BlockSpec((1, dhid), lambda i: (0, 0)),
            pl.BlockSpec((dhid, DC), lambda i: (0, 0)),
        ],
        out_specs=pl.BlockSpec((_BT, DC), lambda i: (i, 0)),
        out_shape=jax.ShapeDtypeStruct((N_NODES, DC), jnp.float32),
    )(degp, s, u, w2, b2, g2, be2, w3)


def _final_body(degp_ref, s_ref, u_ref, b3_ref, batch_ref, out_ref, cnt_ref):
    i = pl.program_id(0)

    @pl.when(i == 0)
    def _init():
        out_ref[...] = jnp.zeros_like(out_ref)
        cnt_ref[...] = jnp.zeros_like(cnt_ref)

    dinv = _dinv_of(degp_ref)
    h = (s_ref[0] + s_ref[1] + u_ref[...]) * dinv[:, None] + b3_ref[0, :]
    bt = batch_ref[0, 0, :]
    gid = lax.broadcasted_iota(jnp.int32, (_BT, N_GRAPHS), 1)
    m = (bt[:, None] == gid).astype(jnp.float32)
    out_ref[...] += lax.dot_general(
        m, h, (((0,), (0,)), ((), ())), preferred_element_type=jnp.float32)
    cnt_ref[0, :] += jnp.sum(m, axis=0)

    @pl.when(i == _NB - 1)
    def _fin():
        out_ref[...] = out_ref[...] / jnp.maximum(cnt_ref[0, :], 1.0)[:, None]


def _tc_final(degp, s, u, b3, batch_r):
    return pl.pallas_call(
        _final_body,
        grid=(_NB,),
        in_specs=[
            pl.BlockSpec((1, 1, _BT), lambda i: (i, 0, 0)),
            pl.BlockSpec((2, _BT, DC), lambda i: (0, i, 0)),
            pl.BlockSpec((_BT, DC), lambda i: (i, 0)),
            pl.BlockSpec((1, DC), lambda i: (0, 0)),
            pl.BlockSpec((1, 1, _BT), lambda i: (i, 0, 0)),
        ],
        out_specs=pl.BlockSpec((N_GRAPHS, DC), lambda i: (0, 0)),
        out_shape=jax.ShapeDtypeStruct((N_GRAPHS, DC), jnp.float32),
        scratch_shapes=[pltpu.VMEM((1, N_GRAPHS), jnp.float32)],
    )(degp, s, u, b3, batch_r)


def kernel(x, edge_index, batch, W1, b1, W2, b2, W3, b3, g1, be1, g2, be2):
    src = edge_index[0].astype(jnp.int32)
    dst = edge_index[1].astype(jnp.int32)

    pad_e = E_PAD_E - N_EDGES
    spread_e = jnp.arange(pad_e, dtype=jnp.int32) % N_PADROWS
    src_es = jnp.concatenate([src, spread_e]).reshape(
        NUM_CORES, NUM_TILES, CHUNKS_E, LANE)
    dst_es = jnp.concatenate([dst, N_NODES + spread_e]).reshape(
        NUM_CORES, NUM_TILES, CHUNKS_E, LANE)

    pad_c = E_PAD_C - N_EDGES
    spread_c = jnp.arange(pad_c, dtype=jnp.int32) % N_PADROWS
    src_p = jnp.concatenate([src, spread_c])
    dst_p = jnp.concatenate([dst, N_NODES + spread_c]).reshape(
        NUM_TILES, CHUNKS_C, LANE)
    src_cs = jnp.stack([src_p, src_p + N_NODES]).reshape(
        NUM_CORES, NUM_TILES, CHUNKS_C, LANE)
    dst_cs = jnp.stack([dst_p, dst_p])

    ones1 = jnp.ones((LANE,), jnp.float32)
    zeros1 = jnp.zeros((N_ACC,), jnp.float32)
    zeros128 = jnp.zeros((N_ACC, DC), jnp.float32)

    b1r, b2r, b3r = b1.reshape(1, -1), b2.reshape(1, -1), b3.reshape(1, -1)
    g1r, be1r = g1.reshape(1, -1), be1.reshape(1, -1)
    g2r, be2r = g2.reshape(1, -1), be2.reshape(1, -1)
    batch_r = batch.astype(jnp.int32).reshape(_NB, 1, _BT)

    degp_raw = _deg_kernel(dst_es, ones1, zeros1).reshape(NUM_CORES, N_ACC)
    degp = (degp_raw[0, :N_NODES] + degp_raw[1, :N_NODES]).reshape(
        _NB, 1, _BT)

    u0 = _tc_scale(degp, x)
    s0 = _prop_edge(u0, src_es, dst_es, zeros128)
    u1 = _tc_layer1(degp, s0, u0, W1, b1r, g1r, be1r)

    s1 = _prop_col(u1.reshape(2 * N_NODES, DC), src_cs, dst_cs, zeros128)
    u2 = _tc_layer3(degp, s1, u1, W2, b2r, g2r, be2r, W3)

    s2 = _prop_edge(u2, src_es, dst_es, zeros128)
    return _tc_final(degp, s2, u2, b3r, batch_r)


# speedup vs baseline: 1.0010x; 1.0010x over previous
import functools

import jax
import jax.numpy as jnp
from jax import lax
from jax.experimental import pallas as pl
from jax.experimental.pallas import tpu as pltpu
from jax.experimental.pallas import tpu_sc as plsc

N_NODES = 10000
N_EDGES = 320000
N_GRAPHS = 16

NUM_CORES = 2
NUM_TILES = 16
LANE = 128
DC = 128
G = 16

ROWS_PER_TILE = 632
N_ACC = ROWS_PER_TILE * NUM_TILES
N_PADROWS = 16

CHUNKS_C = 160
E_PAD_C = NUM_TILES * CHUNKS_C * LANE

CHUNKS_E = 80
E_PAD_E = NUM_CORES * NUM_TILES * CHUNKS_E * LANE

_MESH = plsc.VectorSubcoreMesh(core_axis_name="c", subcore_axis_name="s")


def _make_deg_kernel():
    B = 8
    assert CHUNKS_E % B == 0

    @functools.partial(
        pl.kernel,
        out_type=jax.ShapeDtypeStruct((NUM_CORES * N_ACC,), jnp.float32),
        mesh=_MESH,
        scratch_types=[
            pltpu.VMEM((CHUNKS_E, LANE), jnp.int32),
            pltpu.VMEM((LANE,), jnp.float32),
            pltpu.VMEM((ROWS_PER_TILE,), jnp.float32),
            pltpu.VMEM_SHARED((N_ACC,), jnp.float32),
            pltpu.SemaphoreType.DMA,
        ],
    )
    def deg_kernel(dst_hbm, ones_hbm, zeros_hbm, out_hbm, dst_v, ones_v,
                   tmp_v, acc_sh, sem):
        c = lax.axis_index("c")
        s = lax.axis_index("s")
        row0 = s * ROWS_PER_TILE
        my_rows = pl.ds(row0, ROWS_PER_TILE)
        pltpu.sync_copy(zeros_hbm.at[my_rows], tmp_v)
        pltpu.sync_copy(tmp_v, acc_sh.at[my_rows])
        pltpu.sync_copy(dst_hbm.at[c, s], dst_v)
        pltpu.sync_copy(ones_hbm, ones_v)
        plsc.subcore_barrier()

        @pl.loop(0, CHUNKS_E, step=B)
        def _batch(k0):
            for j in range(B):
                pltpu.async_copy(ones_v, acc_sh.at[dst_v.at[k0 + j]], sem,
                                 add=True)
            for j in range(B):
                pltpu.make_async_copy(
                    ones_v, acc_sh.at[dst_v.at[k0 + j]], sem).wait()

        plsc.subcore_barrier()
        pltpu.sync_copy(acc_sh.at[my_rows], tmp_v)
        pltpu.sync_copy(tmp_v,
                        out_hbm.at[pl.ds(c * N_ACC + row0, ROWS_PER_TILE)])

    return deg_kernel


def _make_prop_kernel(nchunks):
    ngroups = nchunks // G
    assert ngroups * G == nchunks

    @functools.partial(
        pl.kernel,
        out_type=jax.ShapeDtypeStruct((NUM_CORES, N_ACC, DC), jnp.float32),
        mesh=_MESH,
        scratch_types=[
            pltpu.VMEM((2, G, LANE), jnp.int32),
            pltpu.VMEM((2, G, LANE), jnp.int32),
            pltpu.VMEM((2, LANE, DC), jnp.float32),
            pltpu.VMEM_SHARED((N_ACC, DC), jnp.float32),
            pltpu.SemaphoreType.DMA,
            pltpu.SemaphoreType.DMA,
            pltpu.SemaphoreType.DMA,
            pltpu.SemaphoreType.DMA,
            pltpu.SemaphoreType.DMA,
        ],
    )
    def prop_kernel(u_hbm, src_hbm, dst_hbm, zeros_hbm, out_hbm,
                    src_v, dst_v, rows_v, acc_sh, gsem0, gsem1, ssem0, ssem1,
                    isem):
        c = lax.axis_index("c")
        s = lax.axis_index("s")
        row0 = s * ROWS_PER_TILE
        my_rows = pl.ds(row0, ROWS_PER_TILE)
        pltpu.sync_copy(zeros_hbm.at[my_rows], acc_sh.at[my_rows])
        pltpu.sync_copy(src_hbm.at[c, s, pl.ds(0, G)], src_v.at[0])
        pltpu.sync_copy(dst_hbm.at[c, s, pl.ds(0, G)], dst_v.at[0])
        plsc.subcore_barrier()

        gsems = (gsem0, gsem1)
        ssems = (ssem0, ssem1)
        pltpu.async_copy(u_hbm.at[src_v.at[0, 0]], rows_v.at[0], gsem0)

        @pl.loop(0, ngroups)
        def _group(g):
            gi = lax.rem(g, 2)
            for j in range(G):
                k = g * G + j
                b = j % 2

                @pl.when(k > 0)
                def _wait_prev_scatter():
                    pltpu.make_async_copy(
                        rows_v.at[1 - b], acc_sh.at[dst_v.at[gi, j]],
                        ssems[1 - b]).wait()

                if j == 0:
                    @pl.when(g + 1 < ngroups)
                    def _load_next():
                        off = (g + 1) * G
                        pltpu.async_copy(
                            src_hbm.at[c, s, pl.ds(off, G)],
                            src_v.at[1 - gi], isem)
                        pltpu.async_copy(
                            dst_hbm.at[c, s, pl.ds(off, G)],
                            dst_v.at[1 - gi], isem)

                if j == G - 1:
                    @pl.when(g + 1 < ngroups)
                    def _wait_idx():
                        pltpu.make_async_copy(
                            src_hbm.at[c, s, pl.ds(0, G)], src_v.at[0],
                            isem).wait()
                        pltpu.make_async_copy(
                            dst_hbm.at[c, s, pl.ds(0, G)], dst_v.at[0],
                            isem).wait()

                @pl.when(k + 1 < nchunks)
                def _next_gather():
                    if j == G - 1:
                        idx_row = src_v.at[1 - gi, 0]
                    else:
                        idx_row = src_v.at[gi, j + 1]
                    pltpu.async_copy(u_hbm.at[idx_row], rows_v.at[1 - b],
                                     gsems[1 - b])

                pltpu.make_async_copy(
                    u_hbm.at[src_v.at[gi, j]], rows_v.at[b], gsems[b]).wait()
                pltpu.async_copy(rows_v.at[b], acc_sh.at[dst_v.at[gi, j]],
                                 ssems[b], add=True)

        bl = (nchunks - 1) % 2
        pltpu.make_async_copy(
            rows_v.at[bl], acc_sh.at[dst_v.at[0, 0]], ssems[bl]).wait()
        plsc.subcore_barrier()
        pltpu.sync_copy(acc_sh.at[my_rows], out_hbm.at[c, my_rows])

    return prop_kernel


_deg_kernel = _make_deg_kernel()
_prop_edge = _make_prop_kernel(CHUNKS_E)
_prop_col = _make_prop_kernel(CHUNKS_C)


_BT = 1000
_NB = N_NODES // _BT


def _dinv_of(degp_ref):
    deg = degp_ref[0, 0, :] + 1.0
    return lax.rsqrt(jnp.maximum(deg, 1.0))


def _layernorm(h, g_ref, be_ref):
    mu = jnp.mean(h, axis=1, keepdims=True)
    var = jnp.mean((h - mu) ** 2, axis=1, keepdims=True)
    return (h - mu) * lax.rsqrt(var + 1e-5) * g_ref[0, :] + be_ref[0, :]


def _scale_body(degp_ref, x_ref, out_ref):
    dinv = _dinv_of(degp_ref)
    out_ref[...] = x_ref[...] * dinv[:, None]


def _tc_scale(degp, x):
    return pl.pallas_call(
        _scale_body,
        grid=(_NB,),
        in_specs=[
            pl.BlockSpec((1, 1, _BT), lambda i: (i, 0, 0)),
            pl.BlockSpec((_BT, DC), lambda i: (i, 0)),
        ],
        out_specs=pl.BlockSpec((_BT, DC), lambda i: (i, 0)),
        out_shape=jax.ShapeDtypeStruct((N_NODES, DC), jnp.float32),
    )(degp, x)


def _layer1_body(degp_ref, s_ref, u_ref, w_ref, b_ref, g_ref, be_ref, out_ref):
    dinv = _dinv_of(degp_ref)
    a = (s_ref[0] + s_ref[1] + u_ref[...]) * dinv[:, None]
    h = jnp.dot(a, w_ref[...], preferred_element_type=jnp.float32) + b_ref[0, :]
    h = jnp.maximum(h, 0.0)
    un = _layernorm(h, g_ref, be_ref) * dinv[:, None]
    out_ref[0] = un[:, :DC]
    out_ref[1] = un[:, DC:]


def _tc_layer1(degp, s, u, w, b, g, be):
    dhid = w.shape[1]
    return pl.pallas_call(
        _layer1_body,
        grid=(_NB,),
        in_specs=[
            pl.BlockSpec((1, 1, _BT), lambda i: (i, 0, 0)),
            pl.BlockSpec((2, _BT, DC), lambda i: (0, i, 0)),
            pl.BlockSpec((_BT, DC), lambda i: (i, 0)),
            pl.BlockSpec((DC, dhid), lambda i: (0, 0)),
            pl.BlockSpec((1, dhid), lambda i: (0, 0)),
            pl.BlockSpec((1, dhid), lambda i: (0, 0)),
            pl.BlockSpec((1, dhid), lambda i: (0, 0)),
        ],
        out_specs=pl.BlockSpec((2, _BT, DC), lambda i: (0, i, 0)),
        out_shape=jax.ShapeDtypeStruct((2, N_NODES, DC), jnp.float32),
    )(degp, s, u, w, b, g, be)


def _layer3_body(degp_ref, s_ref, u_ref, w2_ref, b2_ref, g2_ref, be2_ref,
                 w3_ref, out_ref):
    dinv = _dinv_of(degp_ref)
    a = jnp.concatenate([s_ref[0] + u_ref[0], s_ref[1] + u_ref[1]], axis=1)
    a = a * dinv[:, None]
    h = jnp.dot(a, w2_ref[...], preferred_element_type=jnp.float32) + b2_ref[0, :]
    h = jnp.maximum(h, 0.0)
    h = _layernorm(h, g2_ref, be2_ref)
    t = jnp.dot(h, w3_ref[...], preferred_element_type=jnp.float32)
    out_ref[...] = t * dinv[:, None]


def _tc_layer3(degp, s, u, w2, b2, g2, be2, w3):
    dhid = w2.shape[1]
    return pl.pallas_call(
        _layer3_body,
        grid=(_NB,),
        in_specs=[
            pl.BlockSpec((1, 1, _BT), lambda i: (i, 0, 0)),
            pl.BlockSpec((2, _BT, DC), lambda i: (0, i, 0)),
            pl.BlockSpec((2, _BT, DC), lambda i: (0, i, 0)),
            pl.BlockSpec((2 * DC, dhid), lambda i: (0, 0)),
            pl.BlockSpec((1, dhid), lambda i: (0, 0)),
            pl.BlockSpec((1, dhid), lambda i: (0, 0)),
            pl.BlockSpec((1, dhid), lambda i: (0, 0)),
            pl.BlockSpec((dhid, DC), lambda i: (0, 0)),
        ],
        out_specs=pl.BlockSpec((_BT, DC), lambda i: (i, 0)),
        out_shape=jax.ShapeDtypeStruct((N_NODES, DC), jnp.float32),
    )(degp, s, u, w2, b2, g2, be2, w3)


def _final_body(degp_ref, s_ref, u_ref, b3_ref, batch_ref, out_ref, cnt_ref):
    i = pl.program_id(0)

    @pl.when(i == 0)
    def _init():
        out_ref[...] = jnp.zeros_like(out_ref)
        cnt_ref[...] = jnp.zeros_like(cnt_ref)

    dinv = _dinv_of(degp_ref)
    h = (s_ref[0] + s_ref[1] + u_ref[...]) * dinv[:, None] + b3_ref[0, :]
    bt = batch_ref[0, 0, :]
    gid = lax.broadcasted_iota(jnp.int32, (_BT, N_GRAPHS), 1)
    m = (bt[:, None] == gid).astype(jnp.float32)
    out_ref[...] += lax.dot_general(
        m, h, (((0,), (0,)), ((), ())), preferred_element_type=jnp.float32)
    cnt_ref[0, :] += jnp.sum(m, axis=0)

    @pl.when(i == _NB - 1)
    def _fin():
        out_ref[...] = out_ref[...] / jnp.maximum(cnt_ref[0, :], 1.0)[:, None]


def _tc_final(degp, s, u, b3, batch_r):
    return pl.pallas_call(
        _final_body,
        grid=(_NB,),
        in_specs=[
            pl.BlockSpec((1, 1, _BT), lambda i: (i, 0, 0)),
            pl.BlockSpec((2, _BT, DC), lambda i: (0, i, 0)),
            pl.BlockSpec((_BT, DC), lambda i: (i, 0)),
            pl.BlockSpec((1, DC), lambda i: (0, 0)),
            pl.BlockSpec((1, 1, _BT), lambda i: (i, 0, 0)),
        ],
        out_specs=pl.BlockSpec((N_GRAPHS, DC), lambda i: (0, 0)),
        out_shape=jax.ShapeDtypeStruct((N_GRAPHS, DC), jnp.float32),
        scratch_shapes=[pltpu.VMEM((1, N_GRAPHS), jnp.float32)],
    )(degp, s, u, b3, batch_r)


def kernel(x, edge_index, batch, W1, b1, W2, b2, W3, b3, g1, be1, g2, be2):
    src = edge_index[0].astype(jnp.int32)
    dst = edge_index[1].astype(jnp.int32)

    pad_e = E_PAD_E - N_EDGES
    spread_e = jnp.arange(pad_e, dtype=jnp.int32) % N_PADROWS
    src_es = jnp.concatenate([src, spread_e]).reshape(
        NUM_CORES, NUM_TILES, CHUNKS_E, LANE)
    dst_es = jnp.concatenate([dst, N_NODES + spread_e]).reshape(
        NUM_CORES, NUM_TILES, CHUNKS_E, LANE)

    pad_c = E_PAD_C - N_EDGES
    spread_c = jnp.arange(pad_c, dtype=jnp.int32) % N_PADROWS
    src_p = jnp.concatenate([src, spread_c])
    dst_p = jnp.concatenate([dst, N_NODES + spread_c]).reshape(
        NUM_TILES, CHUNKS_C, LANE)
    src_cs = jnp.stack([src_p, src_p + N_NODES]).reshape(
        NUM_CORES, NUM_TILES, CHUNKS_C, LANE)
    dst_cs = jnp.stack([dst_p, dst_p])

    ones1 = jnp.ones((LANE,), jnp.float32)
    zeros1 = jnp.zeros((N_ACC,), jnp.float32)
    zeros128 = jnp.zeros((N_ACC, DC), jnp.float32)

    b1r, b2r, b3r = b1.reshape(1, -1), b2.reshape(1, -1), b3.reshape(1, -1)
    g1r, be1r = g1.reshape(1, -1), be1.reshape(1, -1)
    g2r, be2r = g2.reshape(1, -1), be2.reshape(1, -1)
    batch_r = batch.astype(jnp.int32).reshape(_NB, 1, _BT)

    degp_raw = _deg_kernel(dst_es, ones1, zeros1).reshape(NUM_CORES, N_ACC)
    degp = (degp_raw[0, :N_NODES] + degp_raw[1, :N_NODES]).reshape(
        _NB, 1, _BT)

    u0 = _tc_scale(degp, x)
    s0 = _prop_edge(u0, src_es, dst_es, zeros128)
    u1 = _tc_layer1(degp, s0, u0, W1, b1r, g1r, be1r)

    s1 = _prop_col(u1.reshape(2 * N_NODES, DC), src_cs, dst_cs, zeros128)
    u2 = _tc_layer3(degp, s1, u1, W2, b2r, g2r, be2r, W3)

    s2 = _prop_edge(u2, src_es, dst_es, zeros128)
    return _tc_final(degp, s2, u2, b3r, batch_r)


# speedup vs baseline: 1.0081x; 1.0071x over previous
import functools

import jax
import jax.numpy as jnp
from jax import lax
from jax.experimental import pallas as pl
from jax.experimental.pallas import tpu as pltpu
from jax.experimental.pallas import tpu_sc as plsc

N_NODES = 10000
N_EDGES = 320000
N_GRAPHS = 16

NUM_CORES = 2
NUM_TILES = 16
LANE = 128
DC = 128
G = 16

ROWS_PER_TILE = 632
N_ACC = ROWS_PER_TILE * NUM_TILES
N_PADROWS = 112

CHUNKS_C = 160
E_PAD_C = NUM_TILES * CHUNKS_C * LANE

CHUNKS_E = 80
E_PAD_E = NUM_CORES * NUM_TILES * CHUNKS_E * LANE

_MESH = plsc.VectorSubcoreMesh(core_axis_name="c", subcore_axis_name="s")


def _make_deg_kernel():
    B = 8
    assert CHUNKS_E % B == 0

    @functools.partial(
        pl.kernel,
        out_type=jax.ShapeDtypeStruct((NUM_CORES * N_ACC,), jnp.float32),
        mesh=_MESH,
        scratch_types=[
            pltpu.VMEM((CHUNKS_E, LANE), jnp.int32),
            pltpu.VMEM((LANE,), jnp.float32),
            pltpu.VMEM((ROWS_PER_TILE,), jnp.float32),
            pltpu.VMEM_SHARED((N_ACC,), jnp.float32),
            pltpu.SemaphoreType.DMA,
        ],
    )
    def deg_kernel(dst_hbm, ones_hbm, zeros_hbm, out_hbm, dst_v, ones_v,
                   tmp_v, acc_sh, sem):
        c = lax.axis_index("c")
        s = lax.axis_index("s")
        row0 = s * ROWS_PER_TILE
        my_rows = pl.ds(row0, ROWS_PER_TILE)
        pltpu.sync_copy(zeros_hbm.at[my_rows], tmp_v)
        pltpu.sync_copy(tmp_v, acc_sh.at[my_rows])
        pltpu.sync_copy(dst_hbm.at[c, s], dst_v)
        pltpu.sync_copy(ones_hbm, ones_v)
        plsc.subcore_barrier()

        @pl.loop(0, CHUNKS_E, step=B)
        def _batch(k0):
            for j in range(B):
                pltpu.async_copy(ones_v, acc_sh.at[dst_v.at[k0 + j]], sem,
                                 add=True)
            for j in range(B):
                pltpu.make_async_copy(
                    ones_v, acc_sh.at[dst_v.at[k0 + j]], sem).wait()

        plsc.subcore_barrier()
        pltpu.sync_copy(acc_sh.at[my_rows], tmp_v)
        pltpu.sync_copy(tmp_v,
                        out_hbm.at[pl.ds(c * N_ACC + row0, ROWS_PER_TILE)])

    return deg_kernel


def _make_prop_kernel(nchunks):
    ngroups = nchunks // G
    assert ngroups * G == nchunks

    @functools.partial(
        pl.kernel,
        out_type=jax.ShapeDtypeStruct((NUM_CORES, N_ACC, DC), jnp.float32),
        mesh=_MESH,
        scratch_types=[
            pltpu.VMEM((2, G, LANE), jnp.int32),
            pltpu.VMEM((2, G, LANE), jnp.int32),
            pltpu.VMEM((2, LANE, DC), jnp.float32),
            pltpu.VMEM_SHARED((N_ACC, DC), jnp.float32),
            pltpu.SemaphoreType.DMA,
            pltpu.SemaphoreType.DMA,
            pltpu.SemaphoreType.DMA,
            pltpu.SemaphoreType.DMA,
            pltpu.SemaphoreType.DMA,
        ],
    )
    def prop_kernel(u_hbm, src_hbm, dst_hbm, zeros_hbm, out_hbm,
                    src_v, dst_v, rows_v, acc_sh, gsem0, gsem1, ssem0, ssem1,
                    isem):
        c = lax.axis_index("c")
        s = lax.axis_index("s")
        row0 = s * ROWS_PER_TILE
        my_rows = pl.ds(row0, ROWS_PER_TILE)
        pltpu.sync_copy(zeros_hbm.at[my_rows], acc_sh.at[my_rows])
        pltpu.sync_copy(src_hbm.at[c, s, pl.ds(0, G)], src_v.at[0])
        pltpu.sync_copy(dst_hbm.at[c, s, pl.ds(0, G)], dst_v.at[0])
        plsc.subcore_barrier()

        gsems = (gsem0, gsem1)
        ssems = (ssem0, ssem1)
        pltpu.async_copy(u_hbm.at[src_v.at[0, 0]], rows_v.at[0], gsem0)

        @pl.loop(0, ngroups)
        def _group(g):
            gi = lax.rem(g, 2)
            for j in range(G):
                k = g * G + j
                b = j % 2

                @pl.when(k > 0)
                def _wait_prev_scatter():
                    pltpu.make_async_copy(
                        rows_v.at[1 - b], acc_sh.at[dst_v.at[gi, j]],
                        ssems[1 - b]).wait()

                if j == 0:
                    @pl.when(g + 1 < ngroups)
                    def _load_next():
                        off = (g + 1) * G
                        pltpu.async_copy(
                            src_hbm.at[c, s, pl.ds(off, G)],
                            src_v.at[1 - gi], isem)
                        pltpu.async_copy(
                            dst_hbm.at[c, s, pl.ds(off, G)],
                            dst_v.at[1 - gi], isem)

                if j == G - 1:
                    @pl.when(g + 1 < ngroups)
                    def _wait_idx():
                        pltpu.make_async_copy(
                            src_hbm.at[c, s, pl.ds(0, G)], src_v.at[0],
                            isem).wait()
                        pltpu.make_async_copy(
                            dst_hbm.at[c, s, pl.ds(0, G)], dst_v.at[0],
                            isem).wait()

                @pl.when(k + 1 < nchunks)
                def _next_gather():
                    if j == G - 1:
                        idx_row = src_v.at[1 - gi, 0]
                    else:
                        idx_row = src_v.at[gi, j + 1]
                    pltpu.async_copy(u_hbm.at[idx_row], rows_v.at[1 - b],
                                     gsems[1 - b])

                pltpu.make_async_copy(
                    u_hbm.at[src_v.at[gi, j]], rows_v.at[b], gsems[b]).wait()
                pltpu.async_copy(rows_v.at[b], acc_sh.at[dst_v.at[gi, j]],
                                 ssems[b], add=True)

        bl = (nchunks - 1) % 2
        pltpu.make_async_copy(
            rows_v.at[bl], acc_sh.at[dst_v.at[0, 0]], ssems[bl]).wait()
        plsc.subcore_barrier()
        pltpu.sync_copy(acc_sh.at[my_rows], out_hbm.at[c, my_rows])

    return prop_kernel


_deg_kernel = _make_deg_kernel()
_prop_edge = _make_prop_kernel(CHUNKS_E)
_prop_col = _make_prop_kernel(CHUNKS_C)


_BT = 1000
_NB = N_NODES // _BT


def _dinv_of(degp_ref):
    deg = degp_ref[0, 0, :] + 1.0
    return lax.rsqrt(jnp.maximum(deg, 1.0))


def _layernorm(h, g_ref, be_ref):
    mu = jnp.mean(h, axis=1, keepdims=True)
    var = jnp.mean((h - mu) ** 2, axis=1, keepdims=True)
    return (h - mu) * lax.rsqrt(var + 1e-5) * g_ref[0, :] + be_ref[0, :]


def _scale_body(degp_ref, x_ref, out_ref):
    dinv = _dinv_of(degp_ref)
    out_ref[...] = x_ref[...] * dinv[:, None]


def _tc_scale(degp, x):
    return pl.pallas_call(
        _scale_body,
        grid=(_NB,),
        in_specs=[
            pl.BlockSpec((1, 1, _BT), lambda i: (i, 0, 0)),
            pl.BlockSpec((_BT, DC), lambda i: (i, 0)),
        ],
        out_specs=pl.BlockSpec((_BT, DC), lambda i: (i, 0)),
        out_shape=jax.ShapeDtypeStruct((N_NODES, DC), jnp.float32),
    )(degp, x)


def _layer1_body(degp_ref, s_ref, u_ref, w_ref, b_ref, g_ref, be_ref, out_ref):
    dinv = _dinv_of(degp_ref)
    a = (s_ref[0] + s_ref[1] + u_ref[...]) * dinv[:, None]
    h = jnp.dot(a, w_ref[...], preferred_element_type=jnp.float32) + b_ref[0, :]
    h = jnp.maximum(h, 0.0)
    un = _layernorm(h, g_ref, be_ref) * dinv[:, None]
    out_ref[0] = un[:, :DC]
    out_ref[1] = un[:, DC:]


def _tc_layer1(degp, s, u, w, b, g, be):
    dhid = w.shape[1]
    return pl.pallas_call(
        _layer1_body,
        grid=(_NB,),
        in_specs=[
            pl.BlockSpec((1, 1, _BT), lambda i: (i, 0, 0)),
            pl.BlockSpec((2, _BT, DC), lambda i: (0, i, 0)),
            pl.BlockSpec((_BT, DC), lambda i: (i, 0)),
            pl.BlockSpec((DC, dhid), lambda i: (0, 0)),
            pl.BlockSpec((1, dhid), lambda i: (0, 0)),
            pl.BlockSpec((1, dhid), lambda i: (0, 0)),
            pl.BlockSpec((1, dhid), lambda i: (0, 0)),
        ],
        out_specs=pl.BlockSpec((2, _BT, DC), lambda i: (0, i, 0)),
        out_shape=jax.ShapeDtypeStruct((2, N_NODES, DC), jnp.float32),
    )(degp, s, u, w, b, g, be)


def _layer3_body(degp_ref, s_ref, u_ref, w2_ref, b2_ref, g2_ref, be2_ref,
                 w3_ref, out_ref):
    dinv = _dinv_of(degp_ref)
    a = jnp.concatenate([s_ref[0] + u_ref[0], s_ref[1] + u_ref[1]], axis=1)
    a = a * dinv[:, None]
    h = jnp.dot(a, w2_ref[...], preferred_element_type=jnp.float32) + b2_ref[0, :]
    h = jnp.maximum(h, 0.0)
    h = _layernorm(h, g2_ref, be2_ref)
    t = jnp.dot(h, w3_ref[...], preferred_element_type=jnp.float32)
    out_ref[...] = t * dinv[:, None]


def _tc_layer3(degp, s, u, w2, b2, g2, be2, w3):
    dhid = w2.shape[1]
    return pl.pallas_call(
        _layer3_body,
        grid=(_NB,),
        in_specs=[
            pl.BlockSpec((1, 1, _BT), lambda i: (i, 0, 0)),
            pl.BlockSpec((2, _BT, DC), lambda i: (0, i, 0)),
            pl.BlockSpec((2, _BT, DC), lambda i: (0, i, 0)),
            pl.BlockSpec((2 * DC, dhid), lambda i: (0, 0)),
            pl.BlockSpec((1, dhid), lambda i: (0, 0)),
            pl.BlockSpec((1, dhid), lambda i: (0, 0)),
            pl.BlockSpec((1, dhid), lambda i: (0, 0)),
            pl.BlockSpec((dhid, DC), lambda i: (0, 0)),
        ],
        out_specs=pl.BlockSpec((_BT, DC), lambda i: (i, 0)),
        out_shape=jax.ShapeDtypeStruct((N_NODES, DC), jnp.float32),
    )(degp, s, u, w2, b2, g2, be2, w3)


def _final_body(degp_ref, s_ref, u_ref, b3_ref, batch_ref, out_ref, cnt_ref):
    i = pl.program_id(0)

    @pl.when(i == 0)
    def _init():
        out_ref[...] = jnp.zeros_like(out_ref)
        cnt_ref[...] = jnp.zeros_like(cnt_ref)

    dinv = _dinv_of(degp_ref)
    h = (s_ref[0] + s_ref[1] + u_ref[...]) * dinv[:, None] + b3_ref[0, :]
    bt = batch_ref[0, 0, :]
    gid = lax.broadcasted_iota(jnp.int32, (_BT, N_GRAPHS), 1)
    m = (bt[:, None] == gid).astype(jnp.float32)
    out_ref[...] += lax.dot_general(
        m, h, (((0,), (0,)), ((), ())), preferred_element_type=jnp.float32)
    cnt_ref[0, :] += jnp.sum(m, axis=0)

    @pl.when(i == _NB - 1)
    def _fin():
        out_ref[...] = out_ref[...] / jnp.maximum(cnt_ref[0, :], 1.0)[:, None]


def _tc_final(degp, s, u, b3, batch_r):
    return pl.pallas_call(
        _final_body,
        grid=(_NB,),
        in_specs=[
            pl.BlockSpec((1, 1, _BT), lambda i: (i, 0, 0)),
            pl.BlockSpec((2, _BT, DC), lambda i: (0, i, 0)),
            pl.BlockSpec((_BT, DC), lambda i: (i, 0)),
            pl.BlockSpec((1, DC), lambda i: (0, 0)),
            pl.BlockSpec((1, 1, _BT), lambda i: (i, 0, 0)),
        ],
        out_specs=pl.BlockSpec((N_GRAPHS, DC), lambda i: (0, 0)),
        out_shape=jax.ShapeDtypeStruct((N_GRAPHS, DC), jnp.float32),
        scratch_shapes=[pltpu.VMEM((1, N_GRAPHS), jnp.float32)],
    )(degp, s, u, b3, batch_r)


def kernel(x, edge_index, batch, W1, b1, W2, b2, W3, b3, g1, be1, g2, be2):
    src = edge_index[0].astype(jnp.int32)
    dst = edge_index[1].astype(jnp.int32)

    pad_e = E_PAD_E - N_EDGES
    iota_e = jnp.arange(pad_e, dtype=jnp.int32)
    src_es = jnp.concatenate([src, iota_e % N_NODES]).reshape(
        NUM_CORES, NUM_TILES, CHUNKS_E, LANE)
    dst_es = jnp.concatenate([dst, N_NODES + iota_e % N_PADROWS]).reshape(
        NUM_CORES, NUM_TILES, CHUNKS_E, LANE)

    pad_c = E_PAD_C - N_EDGES
    iota_c = jnp.arange(pad_c, dtype=jnp.int32)
    src_p = jnp.concatenate([src, iota_c % N_NODES])
    dst_p = jnp.concatenate([dst, N_NODES + iota_c % N_PADROWS]).reshape(
        NUM_TILES, CHUNKS_C, LANE)
    src_cs = jnp.stack([src_p, src_p + N_NODES]).reshape(
        NUM_CORES, NUM_TILES, CHUNKS_C, LANE)
    dst_cs = jnp.stack([dst_p, dst_p])

    ones1 = jnp.ones((LANE,), jnp.float32)
    zeros1 = jnp.zeros((N_ACC,), jnp.float32)
    zeros128 = jnp.zeros((N_ACC, DC), jnp.float32)

    b1r, b2r, b3r = b1.reshape(1, -1), b2.reshape(1, -1), b3.reshape(1, -1)
    g1r, be1r = g1.reshape(1, -1), be1.reshape(1, -1)
    g2r, be2r = g2.reshape(1, -1), be2.reshape(1, -1)
    batch_r = batch.astype(jnp.int32).reshape(_NB, 1, _BT)

    degp_raw = _deg_kernel(dst_es, ones1, zeros1).reshape(NUM_CORES, N_ACC)
    degp = (degp_raw[0, :N_NODES] + degp_raw[1, :N_NODES]).reshape(
        _NB, 1, _BT)

    u0 = _tc_scale(degp, x)
    s0 = _prop_edge(u0, src_es, dst_es, zeros128)
    u1 = _tc_layer1(degp, s0, u0, W1, b1r, g1r, be1r)

    s1 = _prop_col(u1.reshape(2 * N_NODES, DC), src_cs, dst_cs, zeros128)
    u2 = _tc_layer3(degp, s1, u1, W2, b2r, g2r, be2r, W3)

    s2 = _prop_edge(u2, src_es, dst_es, zeros128)
    return _tc_final(degp, s2, u2, b3r, batch_r)
